# baseline (device time: 187074 ns/iter reference)
import jax
import jax.numpy as jnp
from jax import lax
from jax.experimental import pallas as pl
from jax.experimental.pallas import tpu as pltpu

N_DEV = 4
SQ = 1024
SKV = 1024
NH = 32
NH_LOC = 8
DH = 128
DM = 1024
SCALE = 0.08838834764831843
BLK = 64
NC = 384

CLASSES = [[b for b in range(16) if b % 3 == c] for c in range(3)]
PERM = CLASSES[0] + CLASSES[1] + CLASSES[2]
INV = [PERM.index(b) for b in range(16)]
CLS = [(0, 0, 384), (1, 384, 704), (2, 704, 1024)]
KSETS = [sorted({0} | {b for b in range(16) if b % 3 == r}) for r in range(3)]


def _rows(a, blocks):
    return jnp.concatenate([a[b * BLK:(b + 1) * BLK] for b in blocks], axis=0)


def kernel(x, Wq, K_ext, V_ext, Wo):
    xp = (_rows(x.reshape(SQ, DM), PERM) * SCALE).astype(jnp.bfloat16)
    wq = Wq.astype(jnp.bfloat16)
    wo = Wo.astype(jnp.bfloat16)
    k3 = K_ext.reshape(SKV, NH, DH).astype(jnp.bfloat16)
    v3 = V_ext.reshape(SKV, NH, DH).astype(jnp.bfloat16)
    kc = jnp.stack([_rows(k3, KSETS[r]).transpose(1, 0, 2) for r in range(3)])
    vc = jnp.stack([_rows(v3, KSETS[r]).transpose(1, 0, 2) for r in range(3)])
    kc = kc.reshape(3 * NH, NC, DH)
    vc = vc.reshape(3 * NH, NC, DH)

    def body(x_ref, wq_ref, k_ref, v_ref, wo_ref, out_ref,
             wq_ring, wo_ring, biasd_ref, ctx_ref, ctxu_ref, den_ref,
             cw_send, cw_recv, ccw_send, ccw_recv):
        my = lax.axis_index("i")
        right = lax.rem(my + 1, N_DEV)
        left = lax.rem(my + N_DEV - 1, N_DEV)

        barrier_sem = pltpu.get_barrier_semaphore()
        for nbr in (left, right):
            pl.semaphore_signal(
                barrier_sem, inc=1,
                device_id=(nbr,), device_id_type=pl.DeviceIdType.MESH,
            )
        pl.semaphore_wait(barrier_sem, 2)

        ri = lax.broadcasted_iota(jnp.int32, (320, 320), 0) // BLK
        ci = lax.broadcasted_iota(jnp.int32, (320, 320), 1) // BLK
        biasd_ref[...] = jnp.where(ri == ci, -20.0, -1e9).astype(jnp.bfloat16)

        def attn(h, wq_src, wo_src, first=False):
            j = lax.rem(my + N_DEV - h, N_DEV)
            q = jnp.dot(x_ref[...], wq_src[...],
                        preferred_element_type=jnp.float32)
            qb16 = q.astype(jnp.bfloat16)
            for hh in range(NH_LOC):
                head = j * NH_LOC + hh
                sl = slice(hh * DH, (hh + 1) * DH)
                for (c, r0, r1) in CLS:
                    nr = r1 - r0
                    rc = lax.rem(3 - lax.rem(my + c, 3), 3)
                    idx = rc * NH + head
                    s = lax.dot_general(
                        qb16[r0:r1, sl], k_ref[idx],
                        (((1,), (1,)), ((), ())),
                        preferred_element_type=jnp.float32)
                    w = jnp.exp(s - 20.0)
                    den_ref[0:nr] = jnp.sum(w, axis=1, keepdims=True)
                    ctxu_ref[0:nr] = jnp.dot(
                        w.astype(jnp.bfloat16), v_ref[idx],
                        preferred_element_type=jnp.float32)
                    if c > 0:
                        @pl.when(my == 0)
                        def _(c=c, head=head, r0=r0, r1=r1, nr=nr, sl=sl):
                            idx_d = c * NH + head
                            sd = lax.dot_general(
                                qb16[r0:r1, sl], k_ref[idx_d][BLK:NC],
                                (((1,), (1,)), ((), ())),
                                preferred_element_type=jnp.float32)
                            wd = jnp.exp(
                                sd + biasd_ref[...].astype(jnp.float32))
                            den_ref[0:nr] += jnp.sum(wd, axis=1, keepdims=True)
                            ctxu_ref[0:nr] += jnp.dot(
                                wd.astype(jnp.bfloat16), v_ref[idx_d][BLK:NC],
                                preferred_element_type=jnp.float32)
                    ctxb = (ctxu_ref[0:nr] / den_ref[0:nr]).astype(jnp.bfloat16)
                    if wo_src is None:
                        ctx_ref[r0:r1, sl] = ctxb
                    else:
                        contrib = jnp.dot(
                            ctxb, wo_src[sl, :],
                            preferred_element_type=jnp.float32)
                        if first and hh == 0:
                            out_ref[r0:r1, :] = contrib
                        else:
                            out_ref[r0:r1, :] += contrib

        for h in range(N_DEV - 1):
            cw = pltpu.make_async_remote_copy(
                src_ref=wq_ref if h == 0 else wq_ring.at[h - 1],
                dst_ref=wq_ring.at[h],
                send_sem=cw_send.at[h], recv_sem=cw_recv.at[h],
                device_id=(right,), device_id_type=pl.DeviceIdType.MESH,
            )
            ccw = pltpu.make_async_remote_copy(
                src_ref=wo_ref if h == 0 else wo_ring.at[h - 1],
                dst_ref=wo_ring.at[h],
                send_sem=ccw_send.at[h], recv_sem=ccw_recv.at[h],
                device_id=(left,), device_id_type=pl.DeviceIdType.MESH,
            )
            cw.start()
            ccw.start()
            if h == 0:
                attn(0, wq_ref, wo_ref, first=True)
            elif h == 1:
                attn(1, wq_ring.at[0], None)
            else:
                attn(2, wq_ring.at[1], wo_ring.at[1])
            cw.wait()
            ccw.wait()

        out_ref[...] += jnp.dot(ctx_ref[...], wo_ring[2],
                                preferred_element_type=jnp.float32)
        attn(3, wq_ring.at[2], wo_ring.at[0])

    out = pl.pallas_call(
        body,
        out_shape=jax.ShapeDtypeStruct((SQ, DM), jnp.float32),
        in_specs=[pl.BlockSpec(memory_space=pltpu.VMEM)] * 5,
        out_specs=pl.BlockSpec(memory_space=pltpu.VMEM),
        scratch_shapes=[
            pltpu.VMEM((N_DEV - 1, DM, DM), jnp.bfloat16),
            pltpu.VMEM((N_DEV - 1, DM, DM), jnp.bfloat16),
            pltpu.VMEM((320, 320), jnp.bfloat16),
            pltpu.VMEM((SQ, NH_LOC * DH), jnp.bfloat16),
            pltpu.VMEM((NC, DH), jnp.float32),
            pltpu.VMEM((NC, 1), jnp.float32),
            pltpu.SemaphoreType.DMA((N_DEV - 1,)),
            pltpu.SemaphoreType.DMA((N_DEV - 1,)),
            pltpu.SemaphoreType.DMA((N_DEV - 1,)),
            pltpu.SemaphoreType.DMA((N_DEV - 1,)),
        ],
        compiler_params=pltpu.CompilerParams(
            collective_id=0,
            vmem_limit_bytes=100 * 1024 * 1024,
        ),
    )(xp, wq, kc, vc, wo)
    return _rows(out, INV).reshape(1, SQ, DM)


# device time: 147093 ns/iter; 1.2718x vs baseline; 1.2718x over previous
import jax
import jax.numpy as jnp
from jax import lax
from jax.experimental import pallas as pl
from jax.experimental.pallas import tpu as pltpu

N_DEV = 4
SQ = 1024
SKV = 1024
NH = 32
NH_LOC = 8
DH = 128
DM = 1024
SCALE = 0.08838834764831843


def kernel(x, Wq, K_ext, V_ext, Wo):
    xb = (x.reshape(SQ, DM) * SCALE).astype(jnp.bfloat16)
    wq = Wq.astype(jnp.bfloat16)
    wo = Wo.astype(jnp.bfloat16)
    kt = K_ext.reshape(SKV, NH, DH).astype(jnp.bfloat16).transpose(1, 0, 2)
    vt = V_ext.reshape(SKV, NH, DH).astype(jnp.bfloat16).transpose(1, 0, 2)

    def body(x_ref, wq_ref, k_ref, v_ref, wo_ref, out_ref,
             wq_ring, wo_ring, bias_ref, ctx_ref,
             cw_send, cw_recv, ccw_send, ccw_recv):
        my = lax.axis_index("i")
        right = lax.rem(my + 1, N_DEV)
        left = lax.rem(my + N_DEV - 1, N_DEV)

        barrier_sem = pltpu.get_barrier_semaphore()
        for nbr in (left, right):
            pl.semaphore_signal(
                barrier_sem, inc=1,
                device_id=(nbr,), device_id_type=pl.DeviceIdType.MESH,
            )
        pl.semaphore_wait(barrier_sem, 2)

        rows = lax.broadcasted_iota(jnp.int32, (SQ, SKV), 0)
        cols = lax.broadcasted_iota(jnp.int32, (SQ, SKV), 1)
        qb = (rows + my * SQ) // 64
        kb = cols // 64
        mask = (qb == kb) | (kb == 0) | (lax.rem(qb + kb, 3) == 0)
        bias_ref[...] = jnp.where(mask, -20.0, -1e9).astype(jnp.bfloat16)

        def attn(h, wq_src, wo_src, first=False, half_only=None):
            j = lax.rem(my + N_DEV - h, N_DEV)
            heads = (range(NH_LOC) if half_only is None else
                     range(half_only * NH_LOC // 2,
                           (half_only + 1) * NH_LOC // 2))
            hsl = slice(heads[0] * DH, (heads[-1] + 1) * DH)
            q = jnp.dot(x_ref[...], wq_src[:, hsl],
                        preferred_element_type=jnp.float32)
            qb16 = q.astype(jnp.bfloat16)
            for hh in heads:
                head = j * NH_LOC + hh
                qsl = slice((hh - heads[0]) * DH, (hh - heads[0] + 1) * DH)
                s = lax.dot_general(
                    qb16[:, qsl], k_ref[head],
                    (((1,), (1,)), ((), ())),
                    preferred_element_type=jnp.float32)
                w = jnp.exp(s + bias_ref[...].astype(jnp.float32))
                denom = jnp.sum(w, axis=1, keepdims=True)
                ctxh = jnp.dot(w.astype(jnp.bfloat16), v_ref[head],
                               preferred_element_type=jnp.float32)
                ctxb = (ctxh / denom).astype(jnp.bfloat16)
                if wo_src is None:
                    ctx_ref[:, hh * DH:(hh + 1) * DH] = ctxb
                else:
                    contrib = jnp.dot(
                        ctxb, wo_src[hh * DH:(hh + 1) * DH, :],
                        preferred_element_type=jnp.float32)
                    if first and hh == heads[0]:
                        out_ref[...] = contrib
                    else:
                        out_ref[...] += contrib

        HW = NH_LOC // 2 * DH
        rd = {}
        for h in range(N_DEV - 1):
            for half in range(2):
                csl = slice(half * HW, (half + 1) * HW)
                if h > 0:
                    rd[(h - 1, 0, half)].wait_recv()
                    rd[(h - 1, 1, half)].wait_recv()
                wq_src = wq_ref if h == 0 else wq_ring.at[h - 1]
                wo_src = wo_ref if h == 0 else wo_ring.at[h - 1]
                cw = pltpu.make_async_remote_copy(
                    src_ref=wq_src.at[:, csl],
                    dst_ref=wq_ring.at[h, :, csl],
                    send_sem=cw_send.at[h, half], recv_sem=cw_recv.at[h, half],
                    device_id=(right,), device_id_type=pl.DeviceIdType.MESH,
                )
                ccw = pltpu.make_async_remote_copy(
                    src_ref=wo_src.at[csl, :],
                    dst_ref=wo_ring.at[h, csl, :],
                    send_sem=ccw_send.at[h, half], recv_sem=ccw_recv.at[h, half],
                    device_id=(left,), device_id_type=pl.DeviceIdType.MESH,
                )
                cw.start()
                ccw.start()
                rd[(h, 0, half)] = cw
                rd[(h, 1, half)] = ccw
            if h == 0:
                attn(0, wq_ref, wo_ref, first=True)
            elif h == 1:
                attn(1, wq_ring.at[0], None)
            else:
                attn(2, wq_ring.at[1], wo_ring.at[1])

        for half in range(2):
            rd[(2, 0, half)].wait_recv()
            attn(3, wq_ring.at[2], wo_ring.at[0], half_only=half)
        for half in range(2):
            csl = slice(half * HW, (half + 1) * HW)
            rd[(2, 1, half)].wait_recv()
            out_ref[...] += jnp.dot(ctx_ref[:, csl], wo_ring[2, csl, :],
                                    preferred_element_type=jnp.float32)
        for r in rd.values():
            r.wait_send()

    out = pl.pallas_call(
        body,
        out_shape=jax.ShapeDtypeStruct((SQ, DM), jnp.float32),
        in_specs=[pl.BlockSpec(memory_space=pltpu.VMEM)] * 5,
        out_specs=pl.BlockSpec(memory_space=pltpu.VMEM),
        scratch_shapes=[
            pltpu.VMEM((N_DEV - 1, DM, DM), jnp.bfloat16),
            pltpu.VMEM((N_DEV - 1, DM, DM), jnp.bfloat16),
            pltpu.VMEM((SQ, SKV), jnp.bfloat16),
            pltpu.VMEM((SQ, NH_LOC * DH), jnp.bfloat16),
            pltpu.SemaphoreType.DMA((N_DEV - 1, 2)),
            pltpu.SemaphoreType.DMA((N_DEV - 1, 2)),
            pltpu.SemaphoreType.DMA((N_DEV - 1, 2)),
            pltpu.SemaphoreType.DMA((N_DEV - 1, 2)),
        ],
        compiler_params=pltpu.CompilerParams(
            collective_id=0,
            vmem_limit_bytes=100 * 1024 * 1024,
        ),
    )(xb, wq, kt, vt, wo)
    return out.reshape(1, SQ, DM)
